# baseline (speedup 1.0000x reference)
import functools

import jax
import jax.numpy as jnp
from jax import lax
from jax.experimental import pallas as pl
from jax.experimental.pallas import tpu as pltpu
from jax.experimental.pallas import tpu_sc as plsc

VOCAB = 1000000
EMBED_DIM = 64
PAD_DIM = 128
BATCH = 4096
SEQ = 200

NC = 2
NS = 16
NW = NC * NS
B_BLK = BATCH // NW
N_CHUNKS = SEQ
N_PAIRS = N_CHUNKS // 2
EH = EMBED_DIM // 8


@functools.partial(
    pl.kernel,
    out_type=jax.ShapeDtypeStruct((SEQ, EH, NW, 8, B_BLK), jnp.float32),
    mesh=plsc.VectorSubcoreMesh(core_axis_name="c", subcore_axis_name="s"),
    scratch_types=[
        pltpu.VMEM((N_CHUNKS, B_BLK), jnp.int32),
        pltpu.VMEM((B_BLK, EMBED_DIM), jnp.float32),
        pltpu.VMEM((B_BLK, EMBED_DIM), jnp.float32),
        pltpu.VMEM((EMBED_DIM, B_BLK), jnp.float32),
        pltpu.VMEM((EMBED_DIM, B_BLK), jnp.float32),
        pltpu.SemaphoreType.DMA,
        pltpu.SemaphoreType.DMA,
        pltpu.SemaphoreType.DMA,
        pltpu.SemaphoreType.DMA,
    ],
    compiler_params=pltpu.CompilerParams(
        use_tc_tiling_on_sc=False, needs_layout_passes=False
    ),
)
def _gather_kernel(idx_hbm, table_hbm, out_hbm, idx_v, rows0, rows1,
                   t0, t1, sg0, sg1, ss0, ss1):
    wid = lax.axis_index("s") * NC + lax.axis_index("c")
    rows = (rows0, rows1)
    tbuf = (t0, t1)
    sg = (sg0, sg1)
    ss = (ss0, ss1)

    pltpu.sync_copy(idx_hbm.at[pl.ds(wid * N_CHUNKS, N_CHUNKS)], idx_v)

    lane = lax.iota(jnp.int32, 16)
    rids = [lane + 16 * k for k in range(8)]

    def fire_gather(g, b):
        pltpu.async_copy(table_hbm.at[idx_v.at[g]], rows[b], sg[b])

    def wait_gather(b):
        pltpu.make_async_copy(table_hbm.at[idx_v.at[0]], rows[b],
                              sg[b]).wait()

    one = jnp.full((16,), 1, jnp.int32)

    def transpose(b):
        col = jnp.full((16,), 0, jnp.int32)
        for e in range(EMBED_DIM):
            for k in range(8):
                v = plsc.load_gather(rows[b], [rids[k], col])
                tbuf[b][e, pl.ds(16 * k, 16)] = v
            col = col + one

    def fire_store(g, b):
        for eh in range(EH):
            pltpu.async_copy(tbuf[b].at[pl.ds(8 * eh, 8)],
                             out_hbm.at[g, eh, wid], ss[b])

    def wait_store(b):
        for eh in range(EH):
            pltpu.make_async_copy(tbuf[b].at[pl.ds(8 * eh, 8)],
                                  out_hbm.at[0, eh, wid], ss[b]).wait()

    fire_gather(0, 0)
    fire_gather(1, 1)

    def body(i, carry):
        for b in (0, 1):
            g = 2 * i + b
            wait_gather(b)
            @pl.when(i > 0)
            def _():
                wait_store(b)
            transpose(b)
            fire_store(g, b)
            @pl.when(i < N_PAIRS - 1)
            def _():
                fire_gather(g + 2, b)
        return carry

    lax.fori_loop(0, N_PAIRS, body, 0)
    wait_store(0)
    wait_store(1)


def kernel(input_, weight):
    idx_r = (input_.reshape(NW, B_BLK, SEQ)
             .transpose(0, 2, 1)
             .reshape(NW * SEQ, B_BLK)) * 2
    wpad = jnp.pad(weight, ((0, 0), (0, PAD_DIM - EMBED_DIM)))
    table2 = wpad.reshape(2 * VOCAB, EMBED_DIM)
    out5 = _gather_kernel(idx_r, table2)
    return out5.transpose(2, 4, 0, 1, 3).reshape(BATCH, SEQ, EMBED_DIM)


# speedup vs baseline: 2.7346x; 2.7346x over previous
import functools

import jax
import jax.numpy as jnp
from jax import lax
from jax.experimental import pallas as pl
from jax.experimental.pallas import tpu as pltpu
from jax.experimental.pallas import tpu_sc as plsc

VOCAB = 1000000
EMBED_DIM = 64
PAD_DIM = 128
BATCH = 4096
SEQ = 200

NC = 2
NS = 16
NW = NC * NS
B_BLK = BATCH // NW
N_CHUNKS = SEQ
N_PAIRS = N_CHUNKS // 2
EH = EMBED_DIM // 8


@functools.partial(
    pl.kernel,
    out_type=jax.ShapeDtypeStruct((SEQ, EH, NW, 8, B_BLK), jnp.float32),
    mesh=plsc.VectorSubcoreMesh(core_axis_name="c", subcore_axis_name="s"),
    scratch_types=[
        pltpu.VMEM((N_CHUNKS, B_BLK), jnp.int32),
        pltpu.VMEM((B_BLK, EMBED_DIM), jnp.float32),
        pltpu.VMEM((B_BLK, EMBED_DIM), jnp.float32),
        pltpu.VMEM((EMBED_DIM, B_BLK), jnp.float32),
        pltpu.VMEM((EMBED_DIM, B_BLK), jnp.float32),
        pltpu.SemaphoreType.DMA,
        pltpu.SemaphoreType.DMA,
        pltpu.SemaphoreType.DMA,
        pltpu.SemaphoreType.DMA,
    ],
    compiler_params=pltpu.CompilerParams(
        use_tc_tiling_on_sc=False, needs_layout_passes=False
    ),
)
def _gather_kernel(idx_hbm, table_hbm, out_hbm, idx_v, rows0, rows1,
                   t0, t1, sg0, sg1, ss0, ss1):
    wid = lax.axis_index("s") * NC + lax.axis_index("c")
    rows = (rows0, rows1)
    tbuf = (t0, t1)
    sg = (sg0, sg1)
    ss = (ss0, ss1)

    pltpu.sync_copy(idx_hbm.at[pl.ds(wid * N_CHUNKS, N_CHUNKS)], idx_v)

    lane = lax.iota(jnp.int32, 16)
    rots = []
    _r = lane
    for _ in range(16):
        rots.append(_r)
        _r = jnp.bitwise_and(_r + 1, 15)

    def fire_gather(g, b):
        pltpu.async_copy(table_hbm.at[idx_v.at[g]], rows[b], sg[b])

    def wait_gather(b):
        pltpu.make_async_copy(table_hbm.at[idx_v.at[0]], rows[b],
                              sg[b]).wait()

    def transpose(b):
        @plsc.parallel_loop(0, 32, unroll=2)
        def _(i):
            r0 = (i % 8) * 16
            c0 = (i // 8) * 16
            rowv = lane + r0
            for d in range(16):
                colv = rots[d] + c0
                v = plsc.load_gather(rows[b], [rowv, colv])
                plsc.store_scatter(tbuf[b], [colv, rowv], v)

    def fire_store(g, b):
        for eh in range(EH):
            pltpu.async_copy(tbuf[b].at[pl.ds(8 * eh, 8)],
                             out_hbm.at[g, eh, wid], ss[b])

    def wait_store(b):
        for eh in range(EH):
            pltpu.make_async_copy(tbuf[b].at[pl.ds(8 * eh, 8)],
                                  out_hbm.at[0, eh, wid], ss[b]).wait()

    fire_gather(0, 0)
    fire_gather(1, 1)

    def body(i, carry):
        for b in (0, 1):
            g = 2 * i + b
            wait_gather(b)
            @pl.when(i > 0)
            def _():
                wait_store(b)
            transpose(b)
            fire_store(g, b)
            @pl.when(i < N_PAIRS - 1)
            def _():
                fire_gather(g + 2, b)
        return carry

    lax.fori_loop(0, N_PAIRS, body, 0)
    wait_store(0)
    wait_store(1)


def kernel(input_, weight):
    idx_r = (input_.reshape(NW, B_BLK, SEQ)
             .transpose(0, 2, 1)
             .reshape(NW * SEQ, B_BLK)) * 2
    wpad = jnp.pad(weight, ((0, 0), (0, PAD_DIM - EMBED_DIM)))
    table2 = wpad.reshape(2 * VOCAB, EMBED_DIM)
    out5 = _gather_kernel(idx_r, table2)
    return out5.transpose(2, 4, 0, 1, 3).reshape(BATCH, SEQ, EMBED_DIM)
